# baseline (speedup 1.0000x reference)
import jax
import jax.numpy as jnp
from jax import lax
from jax.experimental import pallas as pl
from jax.experimental.pallas import tpu as pltpu
from jax.experimental.pallas import tpu_sc as plsc

_B, _N, _D = 64, 8192, 64
_NC, _NS, _L = 2, 16, 16
_NW = _NC * _NS
_BPW = _B // _NW
_CHUNK = 512
_CW = _CHUNK * _D
_NCHUNK = _N // _CHUNK
_SLOTS = 8
_RPS = _CHUNK // _SLOTS

_GDN = lax.GatherDimensionNumbers(
    offset_dims=(), collapsed_slice_dims=(0,), start_index_map=(0,))


def _lane_perms(lanes):
    return [jnp.bitwise_xor(lanes, sh)[:, None] for sh in (8, 4, 2, 1)]


def _row_sum_splat(buf, base, perms):
    v0 = buf[pl.ds(base, _L)]
    v1 = buf[pl.ds(base + 16, _L)]
    v2 = buf[pl.ds(base + 32, _L)]
    v3 = buf[pl.ds(base + 48, _L)]
    s = (v0 * v0 + v1 * v1) + (v2 * v2 + v3 * v3)
    for perm in perms:
        s = s + lax.gather(s, perm, _GDN, (1,),
                           mode=lax.GatherScatterMode.PROMISE_IN_BOUNDS)
    return s


def _sc_body(xf_hbm, out_hbm, buf, obuf, iobuf, sem):
    wid = lax.axis_index("s") * _NC + lax.axis_index("c")
    lanes = lax.iota(jnp.int32, _L)
    perms = _lane_perms(lanes)

    for bi in range(_BPW):
        b = wid * _BPW + bi
        row0 = b * _N

        def chunk_body(c, carry):
            pltpu.sync_copy(xf_hbm.at[pl.ds((row0 + c * _CHUNK) * _D, _CW)],
                            buf)

            def jbody(j, carry2):
                ms, ids = carry2
                nms, nids = [], []
                for k in range(_SLOTS):
                    r = c * _CHUNK + k * _RPS + j
                    s = _row_sum_splat(buf, (k * _RPS + j) * _D, perms)
                    upd = s > ms[k]
                    nms.append(jnp.where(upd, s, ms[k]))
                    nids.append(jnp.where(upd, r, ids[k]))
                return (tuple(nms), tuple(nids))

            return lax.fori_loop(0, _RPS, jbody, carry)

        ms0 = tuple(jnp.full((_L,), -1.0, jnp.float32) for _ in range(_SLOTS))
        ids0 = tuple(jnp.zeros((_L,), jnp.int32) for _ in range(_SLOTS))
        ms, ids = lax.fori_loop(0, _NCHUNK, chunk_body, (ms0, ids0))

        m, idxv = ms[0], ids[0]
        for k in range(1, _SLOTS):
            upd = (ms[k] > m) | ((ms[k] == m) & (ids[k] < idxv))
            m = jnp.where(upd, ms[k], m)
            idxv = jnp.where(upd, ids[k], idxv)

        iobuf[pl.ds(0, _L)] = idxv
        widx = iobuf[pl.ds(0, _L)][0]
        pltpu.sync_copy(xf_hbm.at[pl.ds((row0 + widx) * _D, _D)], obuf)
        pltpu.sync_copy(obuf, out_hbm.at[pl.ds(b * _D, _D)])


def kernel(x):
    xf = x.reshape(_B * _N * _D)
    mesh = plsc.VectorSubcoreMesh(core_axis_name="c", subcore_axis_name="s")
    k = pl.kernel(
        _sc_body,
        mesh=mesh,
        out_type=jax.ShapeDtypeStruct((_B * _D,), jnp.float32),
        scratch_types=[
            pltpu.VMEM((_CW,), jnp.float32),
            pltpu.VMEM((_D,), jnp.float32),
            pltpu.VMEM((_L,), jnp.int32),
            pltpu.SemaphoreType.DMA,
        ],
    )
    return k(xf).reshape(_B, 1, _D)
